# baseline (speedup 1.0000x reference)
import functools

import jax
import jax.numpy as jnp
from jax import lax
from jax.experimental import pallas as pl
from jax.experimental.pallas import tpu as pltpu
from jax.experimental.pallas import tpu_sc as plsc

N = 10000
L = 16
SW = 128
NQ = SW // L
NPASS = 4
NSTRIPE = N // SW
CS = 48
NCOL_SC = CS * SW
NCOL_TC = (NSTRIPE - CS) * SW
NW = 32
CHUNK_ROWS = 400
NCHUNK0 = 13
ROWS0 = NCHUNK0 * CHUNK_ROWS
NUNIT = 2 * CS
EXTRA = NUNIT - (NUNIT // NW) * NW
TC_RB = 1000
TC_NRB = N // TC_RB

_GATHER_DNUMS = lax.GatherDimensionNumbers(
    offset_dims=(), collapsed_slice_dims=(0,), start_index_map=(0,))


def _splat_lane(vec, lane):
    idx = jnp.full((L, 1), lane, dtype=jnp.int32)
    return lax.gather(vec, idx, _GATHER_DNUMS, (1,),
                      mode=lax.GatherScatterMode.PROMISE_IN_BOUNDS)


def _body(e_hbm, r_hbm, pval_hbm, pidx_hbm,
          e_v, buf0, buf1, sval_v, sidx_v, sem0, sem1):
    c = lax.axis_index("c")
    s = lax.axis_index("s")
    w = s * 2 + c

    pltpu.sync_copy(e_hbm, e_v)

    def unit_body(i, _):
        u = w + i * NW
        stripe = u // 2
        h = u - 2 * stripe
        c0 = stripe * SW
        r_base = h * ROWS0
        nch = NCHUNK0 - h

        zf = jnp.zeros((L,), jnp.float32)
        zi = jnp.zeros((L,), jnp.int32)
        for q in range(NQ):
            sval_v[pl.ds(q * L, L)] = zf
            sidx_v[pl.ds(q * L, L)] = zi

        def chunk_slice(k):
            return r_hbm.at[pl.ds(r_base + k * CHUNK_ROWS, CHUNK_ROWS),
                            pl.ds(c0, SW)]

        def start(k, buf, sem):
            pltpu.async_copy(chunk_slice(k), buf, sem)

        def wait(k, buf, sem):
            pltpu.make_async_copy(chunk_slice(k), buf, sem).wait()

        def process(k, buf):
            r0 = k * CHUNK_ROWS
            for p in range(NPASS):

                def jbody(j, carry):
                    accs, idxs = carry
                    accs = list(accs)
                    idxs = list(idxs)
                    lbase = j * L
                    gbase = r_base + r0 + lbase
                    e_vec = e_v[pl.ds(gbase, L)]
                    gsplat = jnp.full((L,), gbase, dtype=jnp.int32)
                    for ii in range(L):
                        es = _splat_lane(e_vec, ii)
                        rowv = gsplat + ii
                        for qq in range(NQ // NPASS):
                            q = (NQ // NPASS) * p + qq
                            v = buf[lbase + ii, pl.ds(q * L, L)]
                            pr = v * es
                            m = pr > accs[qq]
                            accs[qq] = jnp.where(m, pr, accs[qq])
                            idxs[qq] = jnp.where(m, rowv, idxs[qq])
                    return (tuple(accs), tuple(idxs))

                qs = [(NQ // NPASS) * p + qq for qq in range(NQ // NPASS)]
                acc0 = tuple(sval_v[pl.ds(q * L, L)] for q in qs)
                idx0 = tuple(sidx_v[pl.ds(q * L, L)] for q in qs)
                accs, idxs = lax.fori_loop(0, CHUNK_ROWS // L, jbody,
                                           (acc0, idx0))
                for qq, q in enumerate(qs):
                    sval_v[pl.ds(q * L, L)] = accs[qq]
                    sidx_v[pl.ds(q * L, L)] = idxs[qq]

        start(0, buf0, sem0)

        def kbody(k, _):
            even = (k % 2) == 0

            @pl.when(k + 1 < nch)
            def _():
                @pl.when(even)
                def _():
                    start(k + 1, buf1, sem1)

                @pl.when(jnp.logical_not(even))
                def _():
                    start(k + 1, buf0, sem0)

            @pl.when(even)
            def _():
                wait(k, buf0, sem0)
                process(k, buf0)

            @pl.when(jnp.logical_not(even))
            def _():
                wait(k, buf1, sem1)
                process(k, buf1)

            return 0

        lax.fori_loop(0, nch, kbody, 0)

        pltpu.sync_copy(sval_v, pval_hbm.at[pl.ds(u * SW, SW)])
        pltpu.sync_copy(sidx_v, pidx_hbm.at[pl.ds(u * SW, SW)])
        return 0

    nunits_w = jnp.where(w < EXTRA, NUNIT // NW + 1, NUNIT // NW)
    lax.fori_loop(0, nunits_w, unit_body, 0)


def _tc_main_body(e_ref, r_ref, val_ref, idx_ref):
    rb = pl.program_id(1)
    p = e_ref[...] * r_ref[...]
    m = jnp.max(p, axis=0, keepdims=True)
    rows = lax.broadcasted_iota(jnp.int32, p.shape, 0) + rb * TC_RB
    cand = jnp.where(p == m, rows, N)
    am = jnp.min(cand, axis=0, keepdims=True)

    @pl.when(rb == 0)
    def _():
        val_ref[...] = m
        idx_ref[...] = am

    @pl.when(rb > 0)
    def _():
        better = m > val_ref[...]
        idx_ref[...] = jnp.where(better, am, idx_ref[...])
        val_ref[...] = jnp.where(better, m, val_ref[...])


def _merge_body(pv_ref, pi_ref, e_ref, r_ref,
                mval_ref, midx_ref, tval_ref, tidx_ref):
    v0 = pv_ref[:, 0, :]
    v1 = pv_ref[:, 1, :]
    i0 = pi_ref[:, 0, :]
    i1 = pi_ref[:, 1, :]
    take1 = v1 > v0
    mval_ref[...] = jnp.where(take1, v1, v0)
    midx_ref[...] = jnp.where(take1, i1, i0)

    p = e_ref[...] * r_ref[...]
    m = jnp.max(p, axis=0, keepdims=True)
    rows = lax.broadcasted_iota(jnp.int32, p.shape, 0)
    cand = jnp.where(p == m, rows, N)
    tval_ref[...] = m
    tidx_ref[...] = jnp.min(cand, axis=0, keepdims=True)


@jax.jit
def _run(e, r):
    mesh = plsc.VectorSubcoreMesh(core_axis_name="c", subcore_axis_name="s")
    sc = functools.partial(
        pl.kernel,
        mesh=mesh,
        out_type=[
            jax.ShapeDtypeStruct((NUNIT * SW,), jnp.float32),
            jax.ShapeDtypeStruct((NUNIT * SW,), jnp.int32),
        ],
        scratch_types=[
            pltpu.VMEM((N,), jnp.float32),
            pltpu.VMEM((CHUNK_ROWS, SW), jnp.float32),
            pltpu.VMEM((CHUNK_ROWS, SW), jnp.float32),
            pltpu.VMEM((SW,), jnp.float32),
            pltpu.VMEM((SW,), jnp.int32),
            pltpu.SemaphoreType.DMA,
            pltpu.SemaphoreType.DMA,
        ],
    )(_body)
    pval, pidx = sc(e, r)

    tc_main = pl.pallas_call(
        _tc_main_body,
        grid=(NCOL_TC // SW, TC_NRB),
        in_specs=[
            pl.BlockSpec((TC_RB, 1), lambda i, j: (j, 0)),
            pl.BlockSpec((TC_RB, SW), lambda i, j: (j, CS + i)),
        ],
        out_specs=[
            pl.BlockSpec((1, SW), lambda i, j: (0, i)),
            pl.BlockSpec((1, SW), lambda i, j: (0, i)),
        ],
        out_shape=[
            jax.ShapeDtypeStruct((1, NCOL_TC), jnp.float32),
            jax.ShapeDtypeStruct((1, NCOL_TC), jnp.int32),
        ],
    )
    cval, cidx = tc_main(e.reshape(N, 1), r)

    tc = pl.pallas_call(
        _merge_body,
        out_shape=[
            jax.ShapeDtypeStruct((CS, SW), jnp.float32),
            jax.ShapeDtypeStruct((CS, SW), jnp.int32),
            jax.ShapeDtypeStruct((1, 16), jnp.float32),
            jax.ShapeDtypeStruct((1, 16), jnp.int32),
        ],
    )
    mval, midx, tval, tidx = tc(
        pval.reshape(CS, 2, SW), pidx.reshape(CS, 2, SW),
        e.reshape(N, 1), r[:, NSTRIPE * SW:])

    val = jnp.concatenate([mval.reshape(NCOL_SC), cval.reshape(NCOL_TC),
                           tval.reshape(16)])
    idx = jnp.concatenate([midx.reshape(NCOL_SC), cidx.reshape(NCOL_TC),
                           tidx.reshape(16)])
    return val.reshape(1, N), idx


def kernel(embedding, r_embedding):
    val, idx = _run(embedding.reshape(N), r_embedding)
    return val, idx


# speedup vs baseline: 1.2210x; 1.2210x over previous
import functools

import jax
import jax.numpy as jnp
from jax import lax
from jax.experimental import pallas as pl
from jax.experimental.pallas import tpu as pltpu
from jax.experimental.pallas import tpu_sc as plsc

N = 10000
L = 16
SW = 128
NQ = SW // L
NPASS = 2
NSTRIPE = N // SW
NCOL_SC = NSTRIPE * SW
NW = 32
CHUNK_ROWS = 400
NCHUNK0 = 13
ROWS0 = NCHUNK0 * CHUNK_ROWS
NUNIT = 2 * NSTRIPE
EXTRA = NUNIT - (NUNIT // NW) * NW

_GATHER_DNUMS = lax.GatherDimensionNumbers(
    offset_dims=(), collapsed_slice_dims=(0,), start_index_map=(0,))


def _splat_lane(vec, lane):
    idx = jnp.full((L, 1), lane, dtype=jnp.int32)
    return lax.gather(vec, idx, _GATHER_DNUMS, (1,),
                      mode=lax.GatherScatterMode.PROMISE_IN_BOUNDS)


def _body(e_hbm, r_hbm, pval_hbm, pidx_hbm,
          e_v, buf0, buf1, sval_v, sidx_v, sem0, sem1):
    c = lax.axis_index("c")
    s = lax.axis_index("s")
    w = s * 2 + c

    pltpu.sync_copy(e_hbm, e_v)

    def unit_body(i, _):
        u = w + i * NW
        h = u // NSTRIPE
        stripe = u - h * NSTRIPE
        c0 = stripe * SW
        r_base = h * ROWS0
        nch = NCHUNK0 - h

        zf = jnp.zeros((L,), jnp.float32)
        zi = jnp.zeros((L,), jnp.int32)
        for q in range(NQ):
            sval_v[pl.ds(q * L, L)] = zf
            sidx_v[pl.ds(q * L, L)] = zi

        def chunk_slice(k):
            return r_hbm.at[pl.ds(r_base + k * CHUNK_ROWS, CHUNK_ROWS),
                            pl.ds(c0, SW)]

        def start(k, buf, sem):
            pltpu.async_copy(chunk_slice(k), buf, sem)

        def wait(k, buf, sem):
            pltpu.make_async_copy(chunk_slice(k), buf, sem).wait()

        def process(k, buf):
            r0 = k * CHUNK_ROWS
            for p in range(NPASS):

                def jbody(j, carry):
                    accs, idxs = carry
                    accs = list(accs)
                    idxs = list(idxs)
                    lbase = j * L
                    gbase = r_base + r0 + lbase
                    e_vec = e_v[pl.ds(gbase, L)]
                    gsplat = jnp.full((L,), gbase, dtype=jnp.int32)
                    for ii in range(L):
                        es = _splat_lane(e_vec, ii)
                        rowv = gsplat + ii
                        for qq in range(NQ // NPASS):
                            q = (NQ // NPASS) * p + qq
                            v = buf[lbase + ii, pl.ds(q * L, L)]
                            pr = v * es
                            m = pr > accs[qq]
                            accs[qq] = jnp.maximum(pr, accs[qq])
                            idxs[qq] = jnp.where(m, rowv, idxs[qq])
                    return (tuple(accs), tuple(idxs))

                qs = [(NQ // NPASS) * p + qq for qq in range(NQ // NPASS)]
                acc0 = tuple(sval_v[pl.ds(q * L, L)] for q in qs)
                idx0 = tuple(sidx_v[pl.ds(q * L, L)] for q in qs)
                accs, idxs = lax.fori_loop(0, CHUNK_ROWS // L, jbody,
                                           (acc0, idx0))
                for qq, q in enumerate(qs):
                    sval_v[pl.ds(q * L, L)] = accs[qq]
                    sidx_v[pl.ds(q * L, L)] = idxs[qq]

        start(0, buf0, sem0)

        def kbody(k, _):
            even = (k % 2) == 0

            @pl.when(k + 1 < nch)
            def _():
                @pl.when(even)
                def _():
                    start(k + 1, buf1, sem1)

                @pl.when(jnp.logical_not(even))
                def _():
                    start(k + 1, buf0, sem0)

            @pl.when(even)
            def _():
                wait(k, buf0, sem0)
                process(k, buf0)

            @pl.when(jnp.logical_not(even))
            def _():
                wait(k, buf1, sem1)
                process(k, buf1)

            return 0

        lax.fori_loop(0, nch, kbody, 0)

        pltpu.sync_copy(sval_v, pval_hbm.at[pl.ds(u * SW, SW)])
        pltpu.sync_copy(sidx_v, pidx_hbm.at[pl.ds(u * SW, SW)])
        return 0

    nunits_w = jnp.where(w < EXTRA, NUNIT // NW + 1, NUNIT // NW)
    lax.fori_loop(0, nunits_w, unit_body, 0)


def _merge_body(pv_ref, pi_ref, e_ref, r_ref,
                mval_ref, midx_ref, tval_ref, tidx_ref):
    v0 = pv_ref[0, :, :]
    v1 = pv_ref[1, :, :]
    i0 = pi_ref[0, :, :]
    i1 = pi_ref[1, :, :]
    take1 = v1 > v0
    mval_ref[...] = jnp.where(take1, v1, v0)
    midx_ref[...] = jnp.where(take1, i1, i0)

    p = e_ref[...] * r_ref[...]
    m = jnp.max(p, axis=0, keepdims=True)
    rows = lax.broadcasted_iota(jnp.int32, p.shape, 0)
    cand = jnp.where(p == m, rows, N)
    tval_ref[...] = m
    tidx_ref[...] = jnp.min(cand, axis=0, keepdims=True)


@jax.jit
def _run(e, r):
    mesh = plsc.VectorSubcoreMesh(core_axis_name="c", subcore_axis_name="s")
    sc = functools.partial(
        pl.kernel,
        mesh=mesh,
        out_type=[
            jax.ShapeDtypeStruct((NUNIT * SW,), jnp.float32),
            jax.ShapeDtypeStruct((NUNIT * SW,), jnp.int32),
        ],
        scratch_types=[
            pltpu.VMEM((N,), jnp.float32),
            pltpu.VMEM((CHUNK_ROWS, SW), jnp.float32),
            pltpu.VMEM((CHUNK_ROWS, SW), jnp.float32),
            pltpu.VMEM((SW,), jnp.float32),
            pltpu.VMEM((SW,), jnp.int32),
            pltpu.SemaphoreType.DMA,
            pltpu.SemaphoreType.DMA,
        ],
    )(_body)
    pval, pidx = sc(e, r)

    tc = pl.pallas_call(
        _merge_body,
        out_shape=[
            jax.ShapeDtypeStruct((NSTRIPE, SW), jnp.float32),
            jax.ShapeDtypeStruct((NSTRIPE, SW), jnp.int32),
            jax.ShapeDtypeStruct((1, 16), jnp.float32),
            jax.ShapeDtypeStruct((1, 16), jnp.int32),
        ],
    )
    mval, midx, tval, tidx = tc(
        pval.reshape(2, NSTRIPE, SW), pidx.reshape(2, NSTRIPE, SW),
        e.reshape(N, 1), r[:, NCOL_SC:])

    val = jnp.concatenate([mval.reshape(NCOL_SC), tval.reshape(16)])
    idx = jnp.concatenate([midx.reshape(NCOL_SC), tidx.reshape(16)])
    return val.reshape(1, N), idx


def kernel(embedding, r_embedding):
    val, idx = _run(embedding.reshape(N), r_embedding)
    return val, idx


# speedup vs baseline: 1.3291x; 1.0886x over previous
import functools

import jax
import jax.numpy as jnp
from jax import lax
from jax.experimental import pallas as pl
from jax.experimental.pallas import tpu as pltpu
from jax.experimental.pallas import tpu_sc as plsc

N = 10000
L = 16
SW = 128
NQ = SW // L
NPASS = 2
NSTRIPE = N // SW
NCOL_SC = NSTRIPE * SW
NW = 32
CHUNK_ROWS = 400
NCHUNK0 = 13
ROWS0 = NCHUNK0 * CHUNK_ROWS
NUNIT = 2 * NSTRIPE
EXTRA = NUNIT - (NUNIT // NW) * NW

_GATHER_DNUMS = lax.GatherDimensionNumbers(
    offset_dims=(), collapsed_slice_dims=(0,), start_index_map=(0,))


def _splat_lane(vec, lane):
    idx = jnp.full((L, 1), lane, dtype=jnp.int32)
    return lax.gather(vec, idx, _GATHER_DNUMS, (1,),
                      mode=lax.GatherScatterMode.PROMISE_IN_BOUNDS)


def _body(e_hbm, r_hbm, pval_hbm, pidx_hbm,
          e_v, buf0, buf1, sval_v, sidx_v, sem0, sem1):
    c = lax.axis_index("c")
    s = lax.axis_index("s")
    w = s * 2 + c

    pltpu.sync_copy(e_hbm, e_v)

    nunits_w = jnp.where(w < EXTRA, NUNIT // NW + 1, NUNIT // NW)

    def unit_slice(u, k):
        h = u // NSTRIPE
        stripe = u - h * NSTRIPE
        return r_hbm.at[pl.ds(h * ROWS0 + k * CHUNK_ROWS, CHUNK_ROWS),
                        pl.ds(stripe * SW, SW)]

    bufs = (buf0, buf1)
    sems = (sem0, sem1)

    pltpu.async_copy(unit_slice(w, 0), buf0, sem0)

    def unit_body(i, phase):
        u = w + i * NW
        h = u // NSTRIPE
        r_base = h * ROWS0
        nch = NCHUNK0 - h

        zf = jnp.zeros((L,), jnp.float32)
        zi = jnp.zeros((L,), jnp.int32)
        for q in range(NQ):
            sval_v[pl.ds(q * L, L)] = zf
            sidx_v[pl.ds(q * L, L)] = zi

        def start(k, b):
            pltpu.async_copy(unit_slice(u, k), bufs[b], sems[b])

        def wait(k, b):
            pltpu.make_async_copy(unit_slice(u, k), bufs[b], sems[b]).wait()

        def process(k, buf):
            r0 = k * CHUNK_ROWS
            for p in range(NPASS):

                def jbody(j, carry):
                    accs, idxs = carry
                    accs = list(accs)
                    idxs = list(idxs)
                    lbase = j * L
                    gbase = r_base + r0 + lbase
                    e_vec = e_v[pl.ds(gbase, L)]
                    gsplat = jnp.full((L,), gbase, dtype=jnp.int32)
                    for ii in range(L):
                        es = _splat_lane(e_vec, ii)
                        rowv = gsplat + ii
                        for qq in range(NQ // NPASS):
                            q = (NQ // NPASS) * p + qq
                            v = buf[lbase + ii, pl.ds(q * L, L)]
                            pr = v * es
                            m = pr > accs[qq]
                            accs[qq] = jnp.maximum(pr, accs[qq])
                            idxs[qq] = jnp.where(m, rowv, idxs[qq])
                    return (tuple(accs), tuple(idxs))

                qs = [(NQ // NPASS) * p + qq for qq in range(NQ // NPASS)]
                acc0 = tuple(sval_v[pl.ds(q * L, L)] for q in qs)
                idx0 = tuple(sidx_v[pl.ds(q * L, L)] for q in qs)
                accs, idxs = lax.fori_loop(0, CHUNK_ROWS // L, jbody,
                                           (acc0, idx0))
                for qq, q in enumerate(qs):
                    sval_v[pl.ds(q * L, L)] = accs[qq]
                    sidx_v[pl.ds(q * L, L)] = idxs[qq]

        def kbody(k, _):
            pk = (k + phase) % 2
            even = pk == 0

            @pl.when(k + 1 < nch)
            def _():
                @pl.when(even)
                def _():
                    start(k + 1, 1)

                @pl.when(jnp.logical_not(even))
                def _():
                    start(k + 1, 0)

            @pl.when(jnp.logical_and(k == nch - 1, i + 1 < nunits_w))
            def _():
                nslice = unit_slice(u + NW, 0)

                @pl.when(even)
                def _():
                    pltpu.async_copy(nslice, buf1, sem1)

                @pl.when(jnp.logical_not(even))
                def _():
                    pltpu.async_copy(nslice, buf0, sem0)

            @pl.when(even)
            def _():
                wait(k, 0)
                process(k, buf0)

            @pl.when(jnp.logical_not(even))
            def _():
                wait(k, 1)
                process(k, buf1)

            return 0

        lax.fori_loop(0, nch, kbody, 0)

        pltpu.sync_copy(sval_v, pval_hbm.at[pl.ds(u * SW, SW)])
        pltpu.sync_copy(sidx_v, pidx_hbm.at[pl.ds(u * SW, SW)])
        return (phase + nch) % 2

    lax.fori_loop(0, nunits_w, unit_body, jnp.int32(0))


def _merge_body(pv_ref, pi_ref, e_ref, r_ref,
                mval_ref, midx_ref, tval_ref, tidx_ref):
    v0 = pv_ref[0, :, :]
    v1 = pv_ref[1, :, :]
    i0 = pi_ref[0, :, :]
    i1 = pi_ref[1, :, :]
    take1 = v1 > v0
    mval_ref[...] = jnp.where(take1, v1, v0)
    midx_ref[...] = jnp.where(take1, i1, i0)

    p = e_ref[...] * r_ref[...]
    m = jnp.max(p, axis=0, keepdims=True)
    rows = lax.broadcasted_iota(jnp.int32, p.shape, 0)
    cand = jnp.where(p == m, rows, N)
    tval_ref[...] = m
    tidx_ref[...] = jnp.min(cand, axis=0, keepdims=True)


@jax.jit
def _run(e, r):
    mesh = plsc.VectorSubcoreMesh(core_axis_name="c", subcore_axis_name="s")
    sc = functools.partial(
        pl.kernel,
        mesh=mesh,
        out_type=[
            jax.ShapeDtypeStruct((NUNIT * SW,), jnp.float32),
            jax.ShapeDtypeStruct((NUNIT * SW,), jnp.int32),
        ],
        scratch_types=[
            pltpu.VMEM((N,), jnp.float32),
            pltpu.VMEM((CHUNK_ROWS, SW), jnp.float32),
            pltpu.VMEM((CHUNK_ROWS, SW), jnp.float32),
            pltpu.VMEM((SW,), jnp.float32),
            pltpu.VMEM((SW,), jnp.int32),
            pltpu.SemaphoreType.DMA,
            pltpu.SemaphoreType.DMA,
        ],
    )(_body)
    pval, pidx = sc(e, r)

    tc = pl.pallas_call(
        _merge_body,
        out_shape=[
            jax.ShapeDtypeStruct((NSTRIPE, SW), jnp.float32),
            jax.ShapeDtypeStruct((NSTRIPE, SW), jnp.int32),
            jax.ShapeDtypeStruct((1, 16), jnp.float32),
            jax.ShapeDtypeStruct((1, 16), jnp.int32),
        ],
    )
    mval, midx, tval, tidx = tc(
        pval.reshape(2, NSTRIPE, SW), pidx.reshape(2, NSTRIPE, SW),
        e.reshape(N, 1), r[:, NCOL_SC:])

    val = jnp.concatenate([mval.reshape(NCOL_SC), tval.reshape(16)])
    idx = jnp.concatenate([midx.reshape(NCOL_SC), tidx.reshape(16)])
    return val.reshape(1, N), idx


def kernel(embedding, r_embedding):
    val, idx = _run(embedding.reshape(N), r_embedding)
    return val, idx
